# Initial kernel scaffold; baseline (speedup 1.0000x reference)
import jax
import jax.numpy as jnp
from jax.experimental import pallas as pl


def kernel(x, edge_index, segment_ids, params):
    raise NotImplementedError("write your pallas kernel here")


# speedup vs baseline: 3.4622x; 3.4622x over previous
import functools

import jax
import jax.numpy as jnp
from jax import lax
from jax.experimental import pallas as pl
from jax.experimental.pallas import tpu as pltpu
from jax.experimental.pallas import tpu_sc as plsc

N = 10000
E = 320000
D = 128
H = 64
G = 64
BN_EPS = 1e-3
EPS = 1e-5

NW = 32
N_PAD = 10240
ROWS_PER_TILE = N_PAD // 16
CHUNK = 128
NCHUNK = 80
EPT = CHUNK * NCHUNK
BLK = 256
NBLK = N_PAD // BLK


def _make_edge_agg(F):
  mesh = plsc.VectorSubcoreMesh(core_axis_name="c", subcore_axis_name="s",
                                num_cores=2, num_subcores=16)

  @functools.partial(
      pl.kernel,
      out_type=jax.ShapeDtypeStruct((2 * N_PAD, F), jnp.float32),
      mesh=mesh,
      scratch_types=[
          pltpu.VMEM((NCHUNK, CHUNK), jnp.int32),
          pltpu.VMEM((NCHUNK, CHUNK), jnp.int32),
          pltpu.VMEM((CHUNK, F), jnp.float32),
          pltpu.VMEM_SHARED((N_PAD, F), jnp.float32),
          pltpu.SemaphoreType.DMA,
      ],
  )
  def edge_kernel(hw_hbm, col_hbm, row_hbm, zeros_hbm, out_hbm,
                  colv, rowv, rowsv, agg, sem):
    c = lax.axis_index("c")
    s = lax.axis_index("s")
    pltpu.sync_copy(zeros_hbm, agg.at[pl.ds(s * ROWS_PER_TILE, ROWS_PER_TILE)])
    base = (c * 16 + s) * NCHUNK
    pltpu.sync_copy(col_hbm.at[pl.ds(base, NCHUNK)], colv)
    pltpu.sync_copy(row_hbm.at[pl.ds(base, NCHUNK)], rowv)
    plsc.subcore_barrier()

    def body(j, carry):
      pltpu.async_copy(hw_hbm.at[colv.at[j]], rowsv, sem).wait()
      pltpu.sync_copy(rowsv, agg.at[rowv.at[j]], add=True)
      return carry

    lax.fori_loop(0, NCHUNK, body, 0)
    plsc.subcore_barrier()
    pltpu.sync_copy(
        agg.at[pl.ds(s * ROWS_PER_TILE, ROWS_PER_TILE)],
        out_hbm.at[pl.ds(c * N_PAD + s * ROWS_PER_TILE, ROWS_PER_TILE)])

  return edge_kernel


_make_edge_agg = functools.lru_cache(maxsize=None)(_make_edge_agg)

F128 = 2 * H


def _edge_agg(hw, col_flat, row_flat, zeros):
  return _make_edge_agg(F128)(hw, col_flat, row_flat, zeros)


def _mm0(xp, w0p):
  def body(x_ref, w_ref, o_ref):
    o_ref[...] = jnp.dot(x_ref[...], w_ref[...],
                         preferred_element_type=jnp.float32)
  return pl.pallas_call(
      body,
      grid=(NBLK,),
      in_specs=[
          pl.BlockSpec((BLK, D), lambda i: (i, 0)),
          pl.BlockSpec((D, F128), lambda i: (0, 0)),
      ],
      out_specs=pl.BlockSpec((BLK, F128), lambda i: (i, 0)),
      out_shape=jax.ShapeDtypeStruct((N_PAD, F128), jnp.float32),
  )(xp, w0p)


def _mm1(agg0, b0p, w1p):
  def body(a_ref, b_ref, w_ref, o_ref):
    h = a_ref[0] + a_ref[1] + b_ref[...]
    h = jnp.maximum(h, 0.0)
    o_ref[...] = jnp.dot(h, w_ref[...], preferred_element_type=jnp.float32)
  return pl.pallas_call(
      body,
      grid=(NBLK,),
      in_specs=[
          pl.BlockSpec((2, BLK, F128), lambda i: (0, i, 0)),
          pl.BlockSpec((1, F128), lambda i: (0, 0)),
          pl.BlockSpec((F128, 2 * H), lambda i: (0, 0)),
      ],
      out_specs=pl.BlockSpec((BLK, 2 * H), lambda i: (i, 0)),
      out_shape=jax.ShapeDtypeStruct((N_PAD, 2 * H), jnp.float32),
  )(agg0.reshape(2, N_PAD, F128), b0p.reshape(1, F128), w1p)


def _pool(xp, agg1, b1, seg3, seg_col):
  F = 2 * H

  def body(x_ref, a_ref, b_ref, s_ref, sc_ref,
           sumh_o, maxh_o, sumx_o, sqx_o, maxx_o, minx_o, cnt_o,
           sumh_s, maxh_s, sumx_s, sqx_s, maxx_s, minx_s, cnt_s):
    i = pl.program_id(0)

    @pl.when(i == 0)
    def _init():
      sumh_s[...] = jnp.zeros_like(sumh_s)
      sumx_s[...] = jnp.zeros_like(sumx_s)
      sqx_s[...] = jnp.zeros_like(sqx_s)
      cnt_s[...] = jnp.zeros_like(cnt_s)
      maxh_s[...] = jnp.full_like(maxh_s, -jnp.inf)
      maxx_s[...] = jnp.full_like(maxx_s, -jnp.inf)
      minx_s[...] = jnp.full_like(minx_s, jnp.inf)

    seg = s_ref[0, 0, :]
    xc = x_ref[...]
    h2 = jnp.maximum(a_ref[0] + a_ref[1] + b_ref[...], 0.0)

    onehot = (seg[None, :] ==
              lax.broadcasted_iota(jnp.int32, (G, BLK), 0)).astype(jnp.float32)
    sumh_s[...] += jnp.dot(onehot, h2, preferred_element_type=jnp.float32)
    sumx_s[...] += jnp.dot(onehot, xc, preferred_element_type=jnp.float32)
    sqx_s[...] += jnp.dot(onehot, xc * xc, preferred_element_type=jnp.float32)
    cnt_s[...] += jnp.sum(onehot, axis=1)[:, None]

    g_lo = seg[0]
    g_hi = jnp.minimum(seg[BLK - 1], G - 1)

    seg_c = sc_ref[...]

    def gbody(g, carry):
      m = (seg_c == g)
      hg = jnp.max(jnp.where(m, h2, -jnp.inf), axis=0)[None, :]
      xg = jnp.max(jnp.where(m, xc, -jnp.inf), axis=0)[None, :]
      ng = jnp.min(jnp.where(m, xc, jnp.inf), axis=0)[None, :]
      maxh_s[pl.ds(g, 1), :] = jnp.maximum(maxh_s[pl.ds(g, 1), :], hg)
      maxx_s[pl.ds(g, 1), :] = jnp.maximum(maxx_s[pl.ds(g, 1), :], xg)
      minx_s[pl.ds(g, 1), :] = jnp.minimum(minx_s[pl.ds(g, 1), :], ng)
      return carry

    lax.fori_loop(g_lo, g_hi + 1, gbody, 0)

    @pl.when(i == NBLK - 1)
    def _write():
      sumh_o[...] = sumh_s[...]
      maxh_o[...] = maxh_s[...]
      sumx_o[...] = sumx_s[...]
      sqx_o[...] = sqx_s[...]
      maxx_o[...] = maxx_s[...]
      minx_o[...] = minx_s[...]
      cnt_o[...] = cnt_s[...]

  out_sds = [jax.ShapeDtypeStruct((G, D), jnp.float32)] * 7
  out_spec = pl.BlockSpec((G, D), lambda i: (0, 0))
  return pl.pallas_call(
      body,
      grid=(NBLK,),
      in_specs=[
          pl.BlockSpec((BLK, D), lambda i: (i, 0)),
          pl.BlockSpec((2, BLK, F), lambda i: (0, i, 0)),
          pl.BlockSpec((1, F), lambda i: (0, 0)),
          pl.BlockSpec((1, 1, BLK), lambda i: (i, 0, 0)),
          pl.BlockSpec((BLK, 1), lambda i: (i, 0)),
      ],
      out_specs=[out_spec] * 7,
      out_shape=out_sds,
      scratch_shapes=[pltpu.VMEM((G, D), jnp.float32)] * 7,
  )(xp, agg1.reshape(2, N_PAD, 2 * H), b1.reshape(1, 2 * H), seg3,
    seg3.reshape(N_PAD, 1))


def _decoder(pooled, params):
  (sumh, maxh, sumx, sqx, maxx, minx, cnt) = pooled
  pnames = []
  for li in range(3):
    pnames += ['dec%d_w' % li, 'dec%d_b' % li, 'bn%d_gamma' % li,
               'bn%d_beta' % li, 'bn%d_mean' % li, 'bn%d_var' % li]
  for hn in ('loge0', 'loge1', 'loge_out', 'ang0', 'ang1', 'ang_out',
             'ang_scale', 'sig0', 'sig1', 'sig_out'):
    pnames += [hn + '_w', hn + '_b']
  pvals = [params[n].reshape(1, -1) if params[n].ndim == 1 else params[n]
           for n in pnames]

  def body(sumh_r, maxh_r, sumx_r, sqx_r, maxx_r, minx_r, cnt_r,
           *refs):
    p = {n: r[...] for n, r in zip(pnames, refs[:-1])}
    o_ref = refs[-1]
    cnt = jnp.maximum(cnt_r[...][:, :1], 1.0)
    sumh = sumh_r[...]
    avg = sumx_r[...] / cnt
    var = jnp.abs(sqx_r[...] / cnt - avg * avg)
    z = jnp.concatenate([maxh_r[...], sumh / cnt, sumh,
                         avg, var, maxx_r[...], minx_r[...]], axis=1)
    for li in range(3):
      z = jnp.dot(z, p['dec%d_w' % li],
                  preferred_element_type=jnp.float32) + p['dec%d_b' % li]
      z = jnp.where(z >= 0, z, 0.15 * z)
      z = ((z - p['bn%d_mean' % li]) *
           lax.rsqrt(p['bn%d_var' % li] + BN_EPS) * p['bn%d_gamma' % li] +
           p['bn%d_beta' % li])

    def dense(t, name):
      return jnp.dot(t, p[name + '_w'],
                     preferred_element_type=jnp.float32) + p[name + '_b']

    x_loge = dense(dense(dense(z, 'loge0'), 'loge1'), 'loge_out')
    x_ang = dense(dense(dense(z, 'ang0'), 'ang1'), 'ang_out')
    zeniazi = jax.nn.sigmoid(dense(x_ang, 'ang_scale'))
    x_sigs = jnp.abs(dense(dense(dense(z, 'sig0'), 'sig1'), 'sig_out')) + EPS
    pi = 3.14159265358979323846
    o_ref[...] = jnp.concatenate(
        [x_loge[:, 0:1], zeniazi[:, 0:1] * pi, zeniazi[:, 1:2] * (2.0 * pi),
         x_sigs], axis=1)

  return pl.pallas_call(
      body,
      out_shape=jax.ShapeDtypeStruct((G, 5), jnp.float32),
  )(sumh, maxh, sumx, sqx, maxx, minx, cnt, *pvals)


def kernel(x, edge_index, segment_ids, params):
  xp = jnp.pad(x, ((0, N_PAD - N), (0, 0)))
  seg3 = jnp.pad(segment_ids, (0, N_PAD - N),
                 constant_values=G).reshape(NBLK, 1, BLK)
  row = edge_index[0].reshape(NW, E // NW)
  col = edge_index[1].reshape(NW, E // NW)
  pad = EPT - E // NW
  row_flat = jnp.pad(row, ((0, 0), (0, pad)),
                     constant_values=N).reshape(NW * NCHUNK, CHUNK)
  col_flat = jnp.pad(col, ((0, 0), (0, pad)),
                     constant_values=0).reshape(NW * NCHUNK, CHUNK)
  z128 = jnp.zeros((ROWS_PER_TILE, F128), jnp.float32)
  w0p = jnp.pad(params['gcn0_w'], ((0, 0), (0, F128 - H)))
  b0p = jnp.pad(params['gcn0_b'], (0, F128 - H))
  w1p = jnp.pad(params['gcn1_w'], ((0, F128 - H), (0, 0)))

  hw0 = _mm0(xp, w0p)
  agg0 = _edge_agg(hw0, col_flat, row_flat, z128)
  hw1 = _mm1(agg0, b0p, w1p)
  agg1 = _edge_agg(hw1, col_flat, row_flat, z128)
  pooled = _pool(xp, agg1, params['gcn1_b'], seg3, seg3)
  return _decoder(pooled, params)


# speedup vs baseline: 4.2083x; 1.2155x over previous
import functools

import jax
import jax.numpy as jnp
from jax import lax
from jax.experimental import pallas as pl
from jax.experimental.pallas import tpu as pltpu
from jax.experimental.pallas import tpu_sc as plsc

N = 10000
E = 320000
D = 128
H = 64
G = 64
BN_EPS = 1e-3
EPS = 1e-5

NW = 32
N_PAD = 10240
ROWS_PER_TILE = N_PAD // 16
CHUNK = 128
NCHUNK = 80
EPT = CHUNK * NCHUNK
BLK = 256
NBLK = N_PAD // BLK
NBUF = 4


def _make_edge_agg(F):
  mesh = plsc.VectorSubcoreMesh(core_axis_name="c", subcore_axis_name="s",
                                num_cores=2, num_subcores=16)

  cp = (pltpu.CompilerParams(use_tc_tiling_on_sc=False) if F == H else None)

  @functools.partial(
      pl.kernel,
      out_type=jax.ShapeDtypeStruct((2 * N_PAD, F), jnp.float32),
      mesh=mesh,
      compiler_params=cp,
      scratch_types=[
          pltpu.VMEM((NCHUNK, CHUNK), jnp.int32),
          pltpu.VMEM((NCHUNK, CHUNK), jnp.int32),
          pltpu.VMEM((CHUNK, F), jnp.float32),
          pltpu.VMEM_SHARED((N_PAD, F), jnp.float32),
          pltpu.SemaphoreType.DMA,
      ],
  )
  def edge_kernel(hw_hbm, col_hbm, row_hbm, zeros_hbm, out_hbm,
                  colv, rowv, ring, agg, sems):
    c = lax.axis_index("c")
    s = lax.axis_index("s")
    pltpu.sync_copy(zeros_hbm, agg.at[pl.ds(s * ROWS_PER_TILE, ROWS_PER_TILE)])
    base = (c * 16 + s) * NCHUNK
    pltpu.sync_copy(col_hbm.at[pl.ds(base, NCHUNK)], colv)
    pltpu.sync_copy(row_hbm.at[pl.ds(base, NCHUNK)], rowv)
    plsc.subcore_barrier()

    def body(j, carry):
      pltpu.async_copy(hw_hbm.at[colv.at[j]], ring, sems).wait()
      pltpu.sync_copy(ring, agg.at[rowv.at[j]], add=True)
      return carry

    lax.fori_loop(0, NCHUNK, body, 0)
    plsc.subcore_barrier()
    pltpu.sync_copy(
        agg.at[pl.ds(s * ROWS_PER_TILE, ROWS_PER_TILE)],
        out_hbm.at[pl.ds(c * N_PAD + s * ROWS_PER_TILE, ROWS_PER_TILE)])

  return edge_kernel


_make_edge_agg = functools.lru_cache(maxsize=None)(_make_edge_agg)

F128 = 2 * H


def _mm0(xp, w0):
  def body(x_ref, w_ref, o_ref):
    o_ref[...] = jnp.dot(x_ref[...], w_ref[...],
                         preferred_element_type=jnp.float32)
  return pl.pallas_call(
      body,
      grid=(NBLK,),
      in_specs=[
          pl.BlockSpec((BLK, D), lambda i: (i, 0)),
          pl.BlockSpec((D, H), lambda i: (0, 0)),
      ],
      out_specs=pl.BlockSpec((BLK, H), lambda i: (i, 0)),
      out_shape=jax.ShapeDtypeStruct((N_PAD, H), jnp.float32),
  )(xp, w0)


def _mm1(agg0, b0, w1):
  def body(a_ref, b_ref, w_ref, o_ref):
    h = a_ref[0] + a_ref[1] + b_ref[...]
    h = jnp.maximum(h, 0.0)
    o_ref[...] = jnp.dot(h, w_ref[...], preferred_element_type=jnp.float32)
  return pl.pallas_call(
      body,
      grid=(NBLK,),
      in_specs=[
          pl.BlockSpec((2, BLK, H), lambda i: (0, i, 0)),
          pl.BlockSpec((1, H), lambda i: (0, 0)),
          pl.BlockSpec((H, 2 * H), lambda i: (0, 0)),
      ],
      out_specs=pl.BlockSpec((BLK, 2 * H), lambda i: (i, 0)),
      out_shape=jax.ShapeDtypeStruct((N_PAD, 2 * H), jnp.float32),
  )(agg0.reshape(2, N_PAD, H), b0.reshape(1, H), w1)


def _pool(xp, agg1, b1, seg3, seg_col):
  F = 2 * H

  def body(x_ref, a_ref, b_ref, s_ref, sc_ref,
           sumh_o, maxh_o, sumx_o, sqx_o, maxx_o, minx_o, cnt_o,
           sumh_s, maxh_s, sumx_s, sqx_s, maxx_s, minx_s, cnt_s):
    i = pl.program_id(0)

    @pl.when(i == 0)
    def _init():
      sumh_s[...] = jnp.zeros_like(sumh_s)
      sumx_s[...] = jnp.zeros_like(sumx_s)
      sqx_s[...] = jnp.zeros_like(sqx_s)
      cnt_s[...] = jnp.zeros_like(cnt_s)
      maxh_s[...] = jnp.full_like(maxh_s, -jnp.inf)
      maxx_s[...] = jnp.full_like(maxx_s, -jnp.inf)
      minx_s[...] = jnp.full_like(minx_s, jnp.inf)

    seg = s_ref[0, 0, :]
    xc = x_ref[...]
    h2 = jnp.maximum(a_ref[0] + a_ref[1] + b_ref[...], 0.0)

    onehot = (seg[None, :] ==
              lax.broadcasted_iota(jnp.int32, (G, BLK), 0)).astype(jnp.float32)
    sumh_s[...] += jnp.dot(onehot, h2, preferred_element_type=jnp.float32)
    sumx_s[...] += jnp.dot(onehot, xc, preferred_element_type=jnp.float32)
    sqx_s[...] += jnp.dot(onehot, xc * xc, preferred_element_type=jnp.float32)
    cnt_s[...] += jnp.sum(onehot, axis=1)[:, None]

    g_lo = seg[0]
    g_hi = jnp.minimum(seg[BLK - 1], G - 1)

    seg_c = sc_ref[...]

    def gbody(g, carry):
      m = (seg_c == g)
      hg = jnp.max(jnp.where(m, h2, -jnp.inf), axis=0)[None, :]
      xg = jnp.max(jnp.where(m, xc, -jnp.inf), axis=0)[None, :]
      ng = jnp.min(jnp.where(m, xc, jnp.inf), axis=0)[None, :]
      maxh_s[pl.ds(g, 1), :] = jnp.maximum(maxh_s[pl.ds(g, 1), :], hg)
      maxx_s[pl.ds(g, 1), :] = jnp.maximum(maxx_s[pl.ds(g, 1), :], xg)
      minx_s[pl.ds(g, 1), :] = jnp.minimum(minx_s[pl.ds(g, 1), :], ng)
      return carry

    lax.fori_loop(g_lo, g_hi + 1, gbody, 0)

    @pl.when(i == NBLK - 1)
    def _write():
      sumh_o[...] = sumh_s[...]
      maxh_o[...] = maxh_s[...]
      sumx_o[...] = sumx_s[...]
      sqx_o[...] = sqx_s[...]
      maxx_o[...] = maxx_s[...]
      minx_o[...] = minx_s[...]
      cnt_o[...] = cnt_s[...]

  out_sds = [jax.ShapeDtypeStruct((G, D), jnp.float32)] * 7
  out_spec = pl.BlockSpec((G, D), lambda i: (0, 0))
  return pl.pallas_call(
      body,
      grid=(NBLK,),
      in_specs=[
          pl.BlockSpec((BLK, D), lambda i: (i, 0)),
          pl.BlockSpec((2, BLK, F), lambda i: (0, i, 0)),
          pl.BlockSpec((1, F), lambda i: (0, 0)),
          pl.BlockSpec((1, 1, BLK), lambda i: (i, 0, 0)),
          pl.BlockSpec((BLK, 1), lambda i: (i, 0)),
      ],
      out_specs=[out_spec] * 7,
      out_shape=out_sds,
      scratch_shapes=[pltpu.VMEM((G, D), jnp.float32)] * 7,
  )(xp, agg1.reshape(2, N_PAD, 2 * H), b1.reshape(1, 2 * H), seg3,
    seg3.reshape(N_PAD, 1))


def _decoder(pooled, params):
  (sumh, maxh, sumx, sqx, maxx, minx, cnt) = pooled
  pnames = []
  for li in range(3):
    pnames += ['dec%d_w' % li, 'dec%d_b' % li, 'bn%d_gamma' % li,
               'bn%d_beta' % li, 'bn%d_mean' % li, 'bn%d_var' % li]
  for hn in ('loge0', 'loge1', 'loge_out', 'ang0', 'ang1', 'ang_out',
             'ang_scale', 'sig0', 'sig1', 'sig_out'):
    pnames += [hn + '_w', hn + '_b']
  pvals = [params[n].reshape(1, -1) if params[n].ndim == 1 else params[n]
           for n in pnames]

  def body(sumh_r, maxh_r, sumx_r, sqx_r, maxx_r, minx_r, cnt_r,
           *refs):
    p = {n: r[...] for n, r in zip(pnames, refs[:-1])}
    o_ref = refs[-1]
    cnt = jnp.maximum(cnt_r[...][:, :1], 1.0)
    sumh = sumh_r[...]
    avg = sumx_r[...] / cnt
    var = jnp.abs(sqx_r[...] / cnt - avg * avg)
    z = jnp.concatenate([maxh_r[...], sumh / cnt, sumh,
                         avg, var, maxx_r[...], minx_r[...]], axis=1)
    for li in range(3):
      z = jnp.dot(z, p['dec%d_w' % li],
                  preferred_element_type=jnp.float32) + p['dec%d_b' % li]
      z = jnp.where(z >= 0, z, 0.15 * z)
      z = ((z - p['bn%d_mean' % li]) *
           lax.rsqrt(p['bn%d_var' % li] + BN_EPS) * p['bn%d_gamma' % li] +
           p['bn%d_beta' % li])

    def dense(t, name):
      return jnp.dot(t, p[name + '_w'],
                     preferred_element_type=jnp.float32) + p[name + '_b']

    x_loge = dense(dense(dense(z, 'loge0'), 'loge1'), 'loge_out')
    x_ang = dense(dense(dense(z, 'ang0'), 'ang1'), 'ang_out')
    zeniazi = jax.nn.sigmoid(dense(x_ang, 'ang_scale'))
    x_sigs = jnp.abs(dense(dense(dense(z, 'sig0'), 'sig1'), 'sig_out')) + EPS
    pi = 3.14159265358979323846
    o_ref[...] = jnp.concatenate(
        [x_loge[:, 0:1], zeniazi[:, 0:1] * pi, zeniazi[:, 1:2] * (2.0 * pi),
         x_sigs], axis=1)

  return pl.pallas_call(
      body,
      out_shape=jax.ShapeDtypeStruct((G, 5), jnp.float32),
  )(sumh, maxh, sumx, sqx, maxx, minx, cnt, *pvals)


def kernel(x, edge_index, segment_ids, params):
  xp = jnp.pad(x, ((0, N_PAD - N), (0, 0)))
  seg3 = jnp.pad(segment_ids, (0, N_PAD - N),
                 constant_values=G).reshape(NBLK, 1, BLK)
  row = edge_index[0].reshape(NW, E // NW)
  col = edge_index[1].reshape(NW, E // NW)
  pad = EPT - E // NW
  row_flat = jnp.pad(row, ((0, 0), (0, pad)),
                     constant_values=N).reshape(NW * NCHUNK, CHUNK)
  col_flat = jnp.pad(col, ((0, 0), (0, pad)),
                     constant_values=0).reshape(NW * NCHUNK, CHUNK)
  z64 = jnp.zeros((ROWS_PER_TILE, H), jnp.float32)
  z128 = jnp.zeros((ROWS_PER_TILE, F128), jnp.float32)

  hw0 = _mm0(xp, params['gcn0_w'])
  agg0 = _make_edge_agg(H)(hw0, col_flat, row_flat, z64)
  hw1 = _mm1(agg0, params['gcn0_b'], params['gcn1_w'])
  agg1 = _make_edge_agg(F128)(hw1, col_flat, row_flat, z128)
  pooled = _pool(xp, agg1, params['gcn1_b'], seg3, seg3)
  return _decoder(pooled, params)


# speedup vs baseline: 4.3482x; 1.0333x over previous
import functools

import jax
import jax.numpy as jnp
from jax import lax
from jax.experimental import pallas as pl
from jax.experimental.pallas import tpu as pltpu
from jax.experimental.pallas import tpu_sc as plsc

N = 10000
E = 320000
D = 128
H = 64
G = 64
BN_EPS = 1e-3
EPS = 1e-5

NW = 32
N_PAD = 10240
ROWS_PER_TILE = N_PAD // 16
CHUNK = 128
NCHUNK = 80
EPT = CHUNK * NCHUNK
BLK = 256
NBLK = N_PAD // BLK
NBUF = 4


def _make_edge_agg(F):
  mesh = plsc.VectorSubcoreMesh(core_axis_name="c", subcore_axis_name="s",
                                num_cores=2, num_subcores=16)

  cp = (pltpu.CompilerParams(use_tc_tiling_on_sc=False) if F == H else None)

  @functools.partial(
      pl.kernel,
      out_type=jax.ShapeDtypeStruct((2 * N_PAD, F), jnp.float32),
      mesh=mesh,
      compiler_params=cp,
      scratch_types=[
          pltpu.VMEM((NCHUNK, CHUNK), jnp.int32),
          pltpu.VMEM((NCHUNK, CHUNK), jnp.int32),
          pltpu.VMEM((CHUNK, F), jnp.float32),
          pltpu.VMEM_SHARED((N_PAD, F), jnp.float32),
          pltpu.SemaphoreType.DMA,
      ],
  )
  def edge_kernel(hw_hbm, col_hbm, row_hbm, zeros_hbm, out_hbm,
                  colv, rowv, ring, agg, sems):
    c = lax.axis_index("c")
    s = lax.axis_index("s")
    pltpu.sync_copy(zeros_hbm, agg.at[pl.ds(s * ROWS_PER_TILE, ROWS_PER_TILE)])
    base = (c * 16 + s) * NCHUNK
    pltpu.sync_copy(col_hbm.at[pl.ds(base, NCHUNK)], colv)
    pltpu.sync_copy(row_hbm.at[pl.ds(base, NCHUNK)], rowv)
    plsc.subcore_barrier()

    def body(j, carry):
      pltpu.async_copy(hw_hbm.at[colv.at[j]], ring, sems).wait()
      pltpu.sync_copy(ring, agg.at[rowv.at[j]], add=True)
      return carry

    lax.fori_loop(0, NCHUNK, body, 0)
    plsc.subcore_barrier()
    pltpu.sync_copy(
        agg.at[pl.ds(s * ROWS_PER_TILE, ROWS_PER_TILE)],
        out_hbm.at[pl.ds(c * N_PAD + s * ROWS_PER_TILE, ROWS_PER_TILE)])

  return edge_kernel


_make_edge_agg = functools.lru_cache(maxsize=None)(_make_edge_agg)


def _make_edge_agg_ring64():
  F = H
  mesh = plsc.VectorSubcoreMesh(core_axis_name="c", subcore_axis_name="s",
                                num_cores=2, num_subcores=16)

  @functools.partial(
      pl.kernel,
      out_type=jax.ShapeDtypeStruct((4 * N_PAD, F), jnp.float32),
      mesh=mesh,
      compiler_params=pltpu.CompilerParams(use_tc_tiling_on_sc=False),
      scratch_types=[
          pltpu.VMEM((NCHUNK, CHUNK), jnp.int32),
          pltpu.VMEM((NCHUNK, CHUNK), jnp.int32),
          pltpu.VMEM((CHUNK, F), jnp.float32),
          pltpu.VMEM((CHUNK, F), jnp.float32),
          pltpu.VMEM_SHARED((N_PAD, F), jnp.float32),
          pltpu.VMEM_SHARED((N_PAD, F), jnp.float32),
          pltpu.SemaphoreType.DMA,
          pltpu.SemaphoreType.DMA,
      ],
  )
  def edge_kernel(hw_hbm, col_hbm, row_hbm, zeros_hbm, out_hbm,
                  colv, rowv, bufa, bufb, agga, aggb, sema, semb):
    c = lax.axis_index("c")
    s = lax.axis_index("s")
    pltpu.sync_copy(zeros_hbm, agga.at[pl.ds(s * ROWS_PER_TILE,
                                             ROWS_PER_TILE)])
    pltpu.sync_copy(zeros_hbm, aggb.at[pl.ds(s * ROWS_PER_TILE,
                                             ROWS_PER_TILE)])
    base = (c * 16 + s) * NCHUNK
    pltpu.sync_copy(col_hbm.at[pl.ds(base, NCHUNK)], colv)
    pltpu.sync_copy(row_hbm.at[pl.ds(base, NCHUNK)], rowv)
    plsc.subcore_barrier()

    pltpu.async_copy(hw_hbm.at[colv.at[0]], bufa, sema)
    pltpu.async_copy(hw_hbm.at[colv.at[1]], bufb, semb)

    def group(k, carry):
      ja = 2 * k
      jb = 2 * k + 1
      pltpu.make_async_copy(hw_hbm.at[colv.at[ja]], bufa, sema).wait()
      pltpu.sync_copy(bufa, agga.at[rowv.at[ja]], add=True)

      @pl.when(ja + 2 < NCHUNK)
      def _():
        pltpu.async_copy(hw_hbm.at[colv.at[ja + 2]], bufa, sema)

      pltpu.make_async_copy(hw_hbm.at[colv.at[jb]], bufb, semb).wait()
      pltpu.sync_copy(bufb, aggb.at[rowv.at[jb]], add=True)

      @pl.when(jb + 2 < NCHUNK)
      def _():
        pltpu.async_copy(hw_hbm.at[colv.at[jb + 2]], bufb, semb)

      return carry

    lax.fori_loop(0, NCHUNK // 2, group, 0)
    plsc.subcore_barrier()
    pltpu.sync_copy(
        agga.at[pl.ds(s * ROWS_PER_TILE, ROWS_PER_TILE)],
        out_hbm.at[pl.ds(2 * c * N_PAD + s * ROWS_PER_TILE, ROWS_PER_TILE)])
    pltpu.sync_copy(
        aggb.at[pl.ds(s * ROWS_PER_TILE, ROWS_PER_TILE)],
        out_hbm.at[pl.ds((2 * c + 1) * N_PAD + s * ROWS_PER_TILE,
                         ROWS_PER_TILE)])

  return edge_kernel


_make_edge_agg_ring64 = functools.lru_cache(maxsize=None)(_make_edge_agg_ring64)

F128 = 2 * H


def _mm0(xp, w0):
  def body(x_ref, w_ref, o_ref):
    o_ref[...] = jnp.dot(x_ref[...], w_ref[...],
                         preferred_element_type=jnp.float32)
  return pl.pallas_call(
      body,
      grid=(NBLK,),
      in_specs=[
          pl.BlockSpec((BLK, D), lambda i: (i, 0)),
          pl.BlockSpec((D, H), lambda i: (0, 0)),
      ],
      out_specs=pl.BlockSpec((BLK, H), lambda i: (i, 0)),
      out_shape=jax.ShapeDtypeStruct((N_PAD, H), jnp.float32),
  )(xp, w0)


def _mm1(agg0, b0, w1):
  def body(a_ref, b_ref, w_ref, o_ref):
    h = a_ref[0] + a_ref[1] + a_ref[2] + a_ref[3] + b_ref[...]
    h = jnp.maximum(h, 0.0)
    o_ref[...] = jnp.dot(h, w_ref[...], preferred_element_type=jnp.float32)
  return pl.pallas_call(
      body,
      grid=(NBLK,),
      in_specs=[
          pl.BlockSpec((4, BLK, H), lambda i: (0, i, 0)),
          pl.BlockSpec((1, H), lambda i: (0, 0)),
          pl.BlockSpec((H, 2 * H), lambda i: (0, 0)),
      ],
      out_specs=pl.BlockSpec((BLK, 2 * H), lambda i: (i, 0)),
      out_shape=jax.ShapeDtypeStruct((N_PAD, 2 * H), jnp.float32),
  )(agg0.reshape(4, N_PAD, H), b0.reshape(1, H), w1)


def _pool(xp, agg1, b1, seg3, seg_col):
  F = 2 * H

  def body(x_ref, a_ref, b_ref, s_ref, sc_ref,
           sumh_o, maxh_o, sumx_o, sqx_o, maxx_o, minx_o, cnt_o,
           sumh_s, maxh_s, sumx_s, sqx_s, maxx_s, minx_s, cnt_s):
    i = pl.program_id(0)

    @pl.when(i == 0)
    def _init():
      sumh_s[...] = jnp.zeros_like(sumh_s)
      sumx_s[...] = jnp.zeros_like(sumx_s)
      sqx_s[...] = jnp.zeros_like(sqx_s)
      cnt_s[...] = jnp.zeros_like(cnt_s)
      maxh_s[...] = jnp.full_like(maxh_s, -jnp.inf)
      maxx_s[...] = jnp.full_like(maxx_s, -jnp.inf)
      minx_s[...] = jnp.full_like(minx_s, jnp.inf)

    seg = s_ref[0, 0, :]
    xc = x_ref[...]
    h2 = jnp.maximum(a_ref[0] + a_ref[1] + b_ref[...], 0.0)

    onehot = (seg[None, :] ==
              lax.broadcasted_iota(jnp.int32, (G, BLK), 0)).astype(jnp.float32)
    sumh_s[...] += jnp.dot(onehot, h2, preferred_element_type=jnp.float32)
    sumx_s[...] += jnp.dot(onehot, xc, preferred_element_type=jnp.float32)
    sqx_s[...] += jnp.dot(onehot, xc * xc, preferred_element_type=jnp.float32)
    cnt_s[...] += jnp.sum(onehot, axis=1)[:, None]

    g_lo = seg[0]
    g_hi = jnp.minimum(seg[BLK - 1], G - 1)

    seg_c = sc_ref[...]

    def gbody(g, carry):
      m = (seg_c == g)
      hg = jnp.max(jnp.where(m, h2, -jnp.inf), axis=0)[None, :]
      xg = jnp.max(jnp.where(m, xc, -jnp.inf), axis=0)[None, :]
      ng = jnp.min(jnp.where(m, xc, jnp.inf), axis=0)[None, :]
      maxh_s[pl.ds(g, 1), :] = jnp.maximum(maxh_s[pl.ds(g, 1), :], hg)
      maxx_s[pl.ds(g, 1), :] = jnp.maximum(maxx_s[pl.ds(g, 1), :], xg)
      minx_s[pl.ds(g, 1), :] = jnp.minimum(minx_s[pl.ds(g, 1), :], ng)
      return carry

    lax.fori_loop(g_lo, g_hi + 1, gbody, 0)

    @pl.when(i == NBLK - 1)
    def _write():
      sumh_o[...] = sumh_s[...]
      maxh_o[...] = maxh_s[...]
      sumx_o[...] = sumx_s[...]
      sqx_o[...] = sqx_s[...]
      maxx_o[...] = maxx_s[...]
      minx_o[...] = minx_s[...]
      cnt_o[...] = cnt_s[...]

  out_sds = [jax.ShapeDtypeStruct((G, D), jnp.float32)] * 7
  out_spec = pl.BlockSpec((G, D), lambda i: (0, 0))
  return pl.pallas_call(
      body,
      grid=(NBLK,),
      in_specs=[
          pl.BlockSpec((BLK, D), lambda i: (i, 0)),
          pl.BlockSpec((2, BLK, F), lambda i: (0, i, 0)),
          pl.BlockSpec((1, F), lambda i: (0, 0)),
          pl.BlockSpec((1, 1, BLK), lambda i: (i, 0, 0)),
          pl.BlockSpec((BLK, 1), lambda i: (i, 0)),
      ],
      out_specs=[out_spec] * 7,
      out_shape=out_sds,
      scratch_shapes=[pltpu.VMEM((G, D), jnp.float32)] * 7,
  )(xp, agg1.reshape(2, N_PAD, 2 * H), b1.reshape(1, 2 * H), seg3,
    seg3.reshape(N_PAD, 1))


def _decoder(pooled, params):
  (sumh, maxh, sumx, sqx, maxx, minx, cnt) = pooled
  pnames = []
  for li in range(3):
    pnames += ['dec%d_w' % li, 'dec%d_b' % li, 'bn%d_gamma' % li,
               'bn%d_beta' % li, 'bn%d_mean' % li, 'bn%d_var' % li]
  for hn in ('loge0', 'loge1', 'loge_out', 'ang0', 'ang1', 'ang_out',
             'ang_scale', 'sig0', 'sig1', 'sig_out'):
    pnames += [hn + '_w', hn + '_b']
  pvals = [params[n].reshape(1, -1) if params[n].ndim == 1 else params[n]
           for n in pnames]

  def body(sumh_r, maxh_r, sumx_r, sqx_r, maxx_r, minx_r, cnt_r,
           *refs):
    p = {n: r[...] for n, r in zip(pnames, refs[:-1])}
    o_ref = refs[-1]
    cnt = jnp.maximum(cnt_r[...][:, :1], 1.0)
    sumh = sumh_r[...]
    avg = sumx_r[...] / cnt
    var = jnp.abs(sqx_r[...] / cnt - avg * avg)
    z = jnp.concatenate([maxh_r[...], sumh / cnt, sumh,
                         avg, var, maxx_r[...], minx_r[...]], axis=1)
    for li in range(3):
      z = jnp.dot(z, p['dec%d_w' % li],
                  preferred_element_type=jnp.float32) + p['dec%d_b' % li]
      z = jnp.where(z >= 0, z, 0.15 * z)
      z = ((z - p['bn%d_mean' % li]) *
           lax.rsqrt(p['bn%d_var' % li] + BN_EPS) * p['bn%d_gamma' % li] +
           p['bn%d_beta' % li])

    def dense(t, name):
      return jnp.dot(t, p[name + '_w'],
                     preferred_element_type=jnp.float32) + p[name + '_b']

    x_loge = dense(dense(dense(z, 'loge0'), 'loge1'), 'loge_out')
    x_ang = dense(dense(dense(z, 'ang0'), 'ang1'), 'ang_out')
    zeniazi = jax.nn.sigmoid(dense(x_ang, 'ang_scale'))
    x_sigs = jnp.abs(dense(dense(dense(z, 'sig0'), 'sig1'), 'sig_out')) + EPS
    pi = 3.14159265358979323846
    o_ref[...] = jnp.concatenate(
        [x_loge[:, 0:1], zeniazi[:, 0:1] * pi, zeniazi[:, 1:2] * (2.0 * pi),
         x_sigs], axis=1)

  return pl.pallas_call(
      body,
      out_shape=jax.ShapeDtypeStruct((G, 5), jnp.float32),
  )(sumh, maxh, sumx, sqx, maxx, minx, cnt, *pvals)


def kernel(x, edge_index, segment_ids, params):
  xp = jnp.pad(x, ((0, N_PAD - N), (0, 0)))
  seg3 = jnp.pad(segment_ids, (0, N_PAD - N),
                 constant_values=G).reshape(NBLK, 1, BLK)
  row = edge_index[0].reshape(NW, E // NW)
  col = edge_index[1].reshape(NW, E // NW)
  pad = EPT - E // NW
  row_flat = jnp.pad(row, ((0, 0), (0, pad)),
                     constant_values=N).reshape(NW * NCHUNK, CHUNK)
  col_flat = jnp.pad(col, ((0, 0), (0, pad)),
                     constant_values=0).reshape(NW * NCHUNK, CHUNK)
  z64 = jnp.zeros((ROWS_PER_TILE, H), jnp.float32)
  z128 = jnp.zeros((ROWS_PER_TILE, F128), jnp.float32)

  hw0 = _mm0(xp, params['gcn0_w'])
  agg0 = _make_edge_agg_ring64()(hw0, col_flat, row_flat, z64)
  hw1 = _mm1(agg0, params['gcn0_b'], params['gcn1_w'])
  agg1 = _make_edge_agg(F128)(hw1, col_flat, row_flat, z128)
  pooled = _pool(xp, agg1, params['gcn1_b'], seg3, seg3)
  return _decoder(pooled, params)


# speedup vs baseline: 4.3489x; 1.0002x over previous
import functools

import jax
import jax.numpy as jnp
from jax import lax
from jax.experimental import pallas as pl
from jax.experimental.pallas import tpu as pltpu
from jax.experimental.pallas import tpu_sc as plsc

N = 10000
E = 320000
D = 128
H = 64
G = 64
BN_EPS = 1e-3
EPS = 1e-5

NW = 32
N_PAD = 10240
ROWS_PER_TILE = N_PAD // 16
CHUNK = 128
NCHUNK = 80
EPT = CHUNK * NCHUNK
BLK = 256
NBLK = N_PAD // BLK


def _make_edge_agg(F):
  mesh = plsc.VectorSubcoreMesh(core_axis_name="c", subcore_axis_name="s",
                                num_cores=2, num_subcores=16)

  cp = (pltpu.CompilerParams(use_tc_tiling_on_sc=False) if F == H else None)

  @functools.partial(
      pl.kernel,
      out_type=jax.ShapeDtypeStruct((2 * N_PAD, F), jnp.float32),
      mesh=mesh,
      compiler_params=cp,
      scratch_types=[
          pltpu.VMEM((NCHUNK, CHUNK), jnp.int32),
          pltpu.VMEM((NCHUNK, CHUNK), jnp.int32),
          pltpu.VMEM((CHUNK, F), jnp.float32),
          pltpu.VMEM_SHARED((N_PAD, F), jnp.float32),
          pltpu.SemaphoreType.DMA,
      ],
  )
  def edge_kernel(hw_hbm, col_hbm, row_hbm, zeros_hbm, out_hbm,
                  colv, rowv, ring, agg, sems):
    c = lax.axis_index("c")
    s = lax.axis_index("s")
    pltpu.sync_copy(zeros_hbm, agg.at[pl.ds(s * ROWS_PER_TILE, ROWS_PER_TILE)])
    base = (c * 16 + s) * NCHUNK
    pltpu.sync_copy(col_hbm.at[pl.ds(base, NCHUNK)], colv)
    pltpu.sync_copy(row_hbm.at[pl.ds(base, NCHUNK)], rowv)
    plsc.subcore_barrier()

    def body(j, carry):
      pltpu.async_copy(hw_hbm.at[colv.at[j]], ring, sems).wait()
      pltpu.sync_copy(ring, agg.at[rowv.at[j]], add=True)
      return carry

    lax.fori_loop(0, NCHUNK, body, 0)
    plsc.subcore_barrier()
    pltpu.sync_copy(
        agg.at[pl.ds(s * ROWS_PER_TILE, ROWS_PER_TILE)],
        out_hbm.at[pl.ds(c * N_PAD + s * ROWS_PER_TILE, ROWS_PER_TILE)])

  return edge_kernel


_make_edge_agg = functools.lru_cache(maxsize=None)(_make_edge_agg)


def _make_edge_agg_ring64():
  F = H
  mesh = plsc.VectorSubcoreMesh(core_axis_name="c", subcore_axis_name="s",
                                num_cores=2, num_subcores=16)

  @functools.partial(
      pl.kernel,
      out_type=jax.ShapeDtypeStruct((4 * N_PAD, F), jnp.float32),
      mesh=mesh,
      compiler_params=pltpu.CompilerParams(use_tc_tiling_on_sc=False),
      scratch_types=[
          pltpu.VMEM((NCHUNK, CHUNK), jnp.int32),
          pltpu.VMEM((NCHUNK, CHUNK), jnp.int32),
          pltpu.VMEM((CHUNK, F), jnp.float32),
          pltpu.VMEM((CHUNK, F), jnp.float32),
          pltpu.VMEM_SHARED((N_PAD, F), jnp.float32),
          pltpu.VMEM_SHARED((N_PAD, F), jnp.float32),
          pltpu.SemaphoreType.DMA,
          pltpu.SemaphoreType.DMA,
      ],
  )
  def edge_kernel(hw_hbm, col_hbm, row_hbm, zeros_hbm, out_hbm,
                  colv, rowv, bufa, bufb, agga, aggb, sema, semb):
    c = lax.axis_index("c")
    s = lax.axis_index("s")
    pltpu.sync_copy(zeros_hbm, agga.at[pl.ds(s * ROWS_PER_TILE,
                                             ROWS_PER_TILE)])
    pltpu.sync_copy(zeros_hbm, aggb.at[pl.ds(s * ROWS_PER_TILE,
                                             ROWS_PER_TILE)])
    base = (c * 16 + s) * NCHUNK
    pltpu.sync_copy(col_hbm.at[pl.ds(base, NCHUNK)], colv)
    pltpu.sync_copy(row_hbm.at[pl.ds(base, NCHUNK)], rowv)
    plsc.subcore_barrier()

    pltpu.async_copy(hw_hbm.at[colv.at[0]], bufa, sema)
    pltpu.async_copy(hw_hbm.at[colv.at[1]], bufb, semb)

    def group(k, carry):
      ja = 2 * k
      jb = 2 * k + 1
      pltpu.make_async_copy(hw_hbm.at[colv.at[ja]], bufa, sema).wait()
      pltpu.sync_copy(bufa, agga.at[rowv.at[ja]], add=True)

      @pl.when(ja + 2 < NCHUNK)
      def _():
        pltpu.async_copy(hw_hbm.at[colv.at[ja + 2]], bufa, sema)

      pltpu.make_async_copy(hw_hbm.at[colv.at[jb]], bufb, semb).wait()
      pltpu.sync_copy(bufb, aggb.at[rowv.at[jb]], add=True)

      @pl.when(jb + 2 < NCHUNK)
      def _():
        pltpu.async_copy(hw_hbm.at[colv.at[jb + 2]], bufb, semb)

      return carry

    lax.fori_loop(0, NCHUNK // 2, group, 0)
    plsc.subcore_barrier()
    pltpu.sync_copy(
        agga.at[pl.ds(s * ROWS_PER_TILE, ROWS_PER_TILE)],
        out_hbm.at[pl.ds(2 * c * N_PAD + s * ROWS_PER_TILE, ROWS_PER_TILE)])
    pltpu.sync_copy(
        aggb.at[pl.ds(s * ROWS_PER_TILE, ROWS_PER_TILE)],
        out_hbm.at[pl.ds((2 * c + 1) * N_PAD + s * ROWS_PER_TILE,
                         ROWS_PER_TILE)])

  return edge_kernel


_make_edge_agg_ring64 = functools.lru_cache(maxsize=None)(_make_edge_agg_ring64)

F128 = 2 * H


def _mm0(xp, w0):
  def body(x_ref, w_ref, o_ref):
    o_ref[...] = jnp.dot(x_ref[...], w_ref[...],
                         preferred_element_type=jnp.float32)
  return pl.pallas_call(
      body,
      grid=(NBLK,),
      in_specs=[
          pl.BlockSpec((BLK, D), lambda i: (i, 0)),
          pl.BlockSpec((D, H), lambda i: (0, 0)),
      ],
      out_specs=pl.BlockSpec((BLK, H), lambda i: (i, 0)),
      out_shape=jax.ShapeDtypeStruct((N_PAD, H), jnp.float32),
  )(xp, w0)


def _mm1(agg0, b0, w1):
  def body(a_ref, b_ref, w_ref, o_ref):
    h = a_ref[0] + a_ref[1] + a_ref[2] + a_ref[3] + b_ref[...]
    h = jnp.maximum(h, 0.0)
    o_ref[...] = jnp.dot(h, w_ref[...], preferred_element_type=jnp.float32)
  return pl.pallas_call(
      body,
      grid=(NBLK,),
      in_specs=[
          pl.BlockSpec((4, BLK, H), lambda i: (0, i, 0)),
          pl.BlockSpec((1, H), lambda i: (0, 0)),
          pl.BlockSpec((H, 2 * H), lambda i: (0, 0)),
      ],
      out_specs=pl.BlockSpec((BLK, 2 * H), lambda i: (i, 0)),
      out_shape=jax.ShapeDtypeStruct((N_PAD, 2 * H), jnp.float32),
  )(agg0.reshape(4, N_PAD, H), b0.reshape(1, H), w1)


def _pool(xp, agg1, b1, seg3, seg_col):
  F = 2 * H

  def body(x_ref, a_ref, b_ref, s_ref, sc_ref,
           sumh_o, maxh_o, sumx_o, sqx_o, maxx_o, minx_o, cnt_o,
           sumh_s, maxh_s, sumx_s, sqx_s, maxx_s, minx_s, cnt_s):
    i = pl.program_id(0)

    @pl.when(i == 0)
    def _init():
      sumh_s[...] = jnp.zeros_like(sumh_s)
      sumx_s[...] = jnp.zeros_like(sumx_s)
      sqx_s[...] = jnp.zeros_like(sqx_s)
      cnt_s[...] = jnp.zeros_like(cnt_s)
      maxh_s[...] = jnp.full_like(maxh_s, -jnp.inf)
      maxx_s[...] = jnp.full_like(maxx_s, -jnp.inf)
      minx_s[...] = jnp.full_like(minx_s, jnp.inf)

    seg = s_ref[0, 0, :]
    xc = x_ref[...]
    h2 = jnp.maximum(a_ref[0] + a_ref[1] + b_ref[...], 0.0)

    onehot = (seg[None, :] ==
              lax.broadcasted_iota(jnp.int32, (G, BLK), 0)).astype(jnp.float32)
    sumh_s[...] += jnp.dot(onehot, h2, preferred_element_type=jnp.float32)
    sumx_s[...] += jnp.dot(onehot, xc, preferred_element_type=jnp.float32)
    sqx_s[...] += jnp.dot(onehot, xc * xc, preferred_element_type=jnp.float32)
    cnt_s[...] += jnp.sum(onehot, axis=1)[:, None]

    g_lo = seg[0]
    g_hi = jnp.minimum(seg[BLK - 1], G - 1)

    seg_c = sc_ref[...]

    def gbody(g, carry):
      m = (seg_c == g)
      hg = jnp.max(jnp.where(m, h2, -jnp.inf), axis=0)[None, :]
      xg = jnp.max(jnp.where(m, xc, -jnp.inf), axis=0)[None, :]
      ng = jnp.min(jnp.where(m, xc, jnp.inf), axis=0)[None, :]
      maxh_s[pl.ds(g, 1), :] = jnp.maximum(maxh_s[pl.ds(g, 1), :], hg)
      maxx_s[pl.ds(g, 1), :] = jnp.maximum(maxx_s[pl.ds(g, 1), :], xg)
      minx_s[pl.ds(g, 1), :] = jnp.minimum(minx_s[pl.ds(g, 1), :], ng)
      return carry

    lax.fori_loop(g_lo, g_hi + 1, gbody, 0)

    @pl.when(i == NBLK - 1)
    def _write():
      sumh_o[...] = sumh_s[...]
      maxh_o[...] = maxh_s[...]
      sumx_o[...] = sumx_s[...]
      sqx_o[...] = sqx_s[...]
      maxx_o[...] = maxx_s[...]
      minx_o[...] = minx_s[...]
      cnt_o[...] = cnt_s[...]

  out_sds = [jax.ShapeDtypeStruct((G, D), jnp.float32)] * 7
  out_spec = pl.BlockSpec((G, D), lambda i: (0, 0))
  return pl.pallas_call(
      body,
      grid=(NBLK,),
      in_specs=[
          pl.BlockSpec((BLK, D), lambda i: (i, 0)),
          pl.BlockSpec((2, BLK, F), lambda i: (0, i, 0)),
          pl.BlockSpec((1, F), lambda i: (0, 0)),
          pl.BlockSpec((1, 1, BLK), lambda i: (i, 0, 0)),
          pl.BlockSpec((BLK, 1), lambda i: (i, 0)),
      ],
      out_specs=[out_spec] * 7,
      out_shape=out_sds,
      scratch_shapes=[pltpu.VMEM((G, D), jnp.float32)] * 7,
  )(xp, agg1.reshape(2, N_PAD, 2 * H), b1.reshape(1, 2 * H), seg3,
    seg3.reshape(N_PAD, 1))


def _decoder(pooled, params):
  (sumh, maxh, sumx, sqx, maxx, minx, cnt) = pooled
  pnames = []
  for li in range(3):
    pnames += ['dec%d_w' % li, 'dec%d_b' % li, 'bn%d_gamma' % li,
               'bn%d_beta' % li, 'bn%d_mean' % li, 'bn%d_var' % li]
  for hn in ('loge0', 'loge1', 'loge_out', 'ang0', 'ang1', 'ang_out',
             'ang_scale', 'sig0', 'sig1', 'sig_out'):
    pnames += [hn + '_w', hn + '_b']
  pvals = [params[n].reshape(1, -1) if params[n].ndim == 1 else params[n]
           for n in pnames]

  def body(sumh_r, maxh_r, sumx_r, sqx_r, maxx_r, minx_r, cnt_r,
           *refs):
    p = {n: r[...] for n, r in zip(pnames, refs[:-1])}
    o_ref = refs[-1]
    cnt = jnp.maximum(cnt_r[...][:, :1], 1.0)
    sumh = sumh_r[...]
    avg = sumx_r[...] / cnt
    var = jnp.abs(sqx_r[...] / cnt - avg * avg)
    z = jnp.concatenate([maxh_r[...], sumh / cnt, sumh,
                         avg, var, maxx_r[...], minx_r[...]], axis=1)
    for li in range(3):
      z = jnp.dot(z, p['dec%d_w' % li],
                  preferred_element_type=jnp.float32) + p['dec%d_b' % li]
      z = jnp.where(z >= 0, z, 0.15 * z)
      z = ((z - p['bn%d_mean' % li]) *
           lax.rsqrt(p['bn%d_var' % li] + BN_EPS) * p['bn%d_gamma' % li] +
           p['bn%d_beta' % li])

    def dense(t, name):
      return jnp.dot(t, p[name + '_w'],
                     preferred_element_type=jnp.float32) + p[name + '_b']

    x_loge = dense(dense(dense(z, 'loge0'), 'loge1'), 'loge_out')
    x_ang = dense(dense(dense(z, 'ang0'), 'ang1'), 'ang_out')
    zeniazi = jax.nn.sigmoid(dense(x_ang, 'ang_scale'))
    x_sigs = jnp.abs(dense(dense(dense(z, 'sig0'), 'sig1'), 'sig_out')) + EPS
    pi = 3.14159265358979323846
    o_ref[...] = jnp.concatenate(
        [x_loge[:, 0:1], zeniazi[:, 0:1] * pi, zeniazi[:, 1:2] * (2.0 * pi),
         x_sigs], axis=1)

  return pl.pallas_call(
      body,
      out_shape=jax.ShapeDtypeStruct((G, 5), jnp.float32),
  )(sumh, maxh, sumx, sqx, maxx, minx, cnt, *pvals)


def kernel(x, edge_index, segment_ids, params):
  xp = jnp.pad(x, ((0, N_PAD - N), (0, 0)))
  seg3 = jnp.pad(segment_ids, (0, N_PAD - N),
                 constant_values=G).reshape(NBLK, 1, BLK)
  row = edge_index[0].reshape(NW, E // NW)
  col = edge_index[1].reshape(NW, E // NW)
  pad = EPT - E // NW
  row_flat = jnp.pad(row, ((0, 0), (0, pad)),
                     constant_values=N).reshape(NW * NCHUNK, CHUNK)
  col_flat = jnp.pad(col, ((0, 0), (0, pad)),
                     constant_values=0).reshape(NW * NCHUNK, CHUNK)
  z64 = jnp.zeros((ROWS_PER_TILE, H), jnp.float32)
  z128 = jnp.zeros((ROWS_PER_TILE, F128), jnp.float32)

  hw0 = _mm0(xp, params['gcn0_w'])
  agg0 = _make_edge_agg_ring64()(hw0, col_flat, row_flat, z64)
  hw1 = _mm1(agg0, params['gcn0_b'], params['gcn1_w'])
  agg1 = _make_edge_agg(F128)(hw1, col_flat, row_flat, z128)
  pooled = _pool(xp, agg1, params['gcn1_b'], seg3, seg3)
  return _decoder(pooled, params)
